# speedup vs baseline: 36.7298x; 36.7298x over previous
import jax
import jax.numpy as jnp
from jax.experimental import pallas as pl
from jax.experimental.pallas import tpu as pltpu

_NUM_HEADS = 32
_MAX_DIST = 10
_NVALS = _MAX_DIST + 2
_H_BLK = 8


def _body(dist_ref, et_ref, out_ref):
    idx = jnp.clip(dist_ref[0] + 1, 0, _NVALS - 1)
    masks = [idx == k for k in range(_NVALS)]
    for hh in range(_H_BLK):
        acc = jnp.broadcast_to(et_ref[hh, _NVALS - 1], idx.shape)
        for k in range(_NVALS - 2, -1, -1):
            acc = jnp.where(masks[k], et_ref[hh, k], acc)
        out_ref[hh] = acc


def kernel(dist_matrix, bias_embedding):
    B, N, _ = dist_matrix.shape
    H = _NUM_HEADS
    nhc = H // _H_BLK
    et = bias_embedding.T

    out = pl.pallas_call(
        _body,
        grid=(B, nhc),
        in_specs=[
            pl.BlockSpec((1, N, N), lambda b, hc: (b, 0, 0)),
            pl.BlockSpec((_H_BLK, _NVALS), lambda b, hc: (hc, 0)),
        ],
        out_specs=pl.BlockSpec((_H_BLK, N, N), lambda b, hc: (b * nhc + hc, 0, 0)),
        out_shape=jax.ShapeDtypeStruct((B * H, N, N), jnp.float32),
    )(dist_matrix, et)
    return out
